# baseline (speedup 1.0000x reference)
import functools

import jax
import jax.numpy as jnp
from jax import lax
from jax.experimental import pallas as pl
from jax.experimental.pallas import tpu as pltpu
from jax.experimental.pallas import tpu_sc as plsc

B, S, D = 1, 2048, 1024
H, DH = 16, 64
QC, KC = 512, 256
E, K, F = 64, 2, 512
CAP = 128
EPS = 1e-6
RES_MULT = 0.22
SCALE = 0.125
T = B * S

RB = 256
QB = 256
KB = 256
NBUF = E * CAP + CAP
DUMMY = E * CAP
CB = 256

NW = 32
CHUNK = T // NW


def _mm_nt(a, b):
    return lax.dot_general(a.astype(jnp.bfloat16), b.astype(jnp.bfloat16),
                           (((1,), (1,)), ((), ())),
                           preferred_element_type=jnp.float32)


def _mm_nn(a, b):
    return lax.dot_general(a.astype(jnp.bfloat16), b.astype(jnp.bfloat16),
                           (((1,), (0,)), ((), ())),
                           preferred_element_type=jnp.float32)


def _preattn_body(x_ref, ln1_ref, wd_ref, wq_ref, wk_ref, wv_ref,
                  q_ref, k_ref, v_ref):
    x = x_ref[...]
    h = x * lax.rsqrt(jnp.mean(x * x, axis=-1, keepdims=True) + EPS) * ln1_ref[...]
    c = _mm_nt(h, wd_ref[...])
    q_ref[...] = _mm_nt(c[:, :QC], wq_ref[...]).astype(jnp.bfloat16)
    k_ref[...] = _mm_nt(c[:, QC:QC + KC], wk_ref[...]).astype(jnp.bfloat16)
    v_ref[...] = _mm_nt(c[:, QC + KC:], wv_ref[...]).astype(jnp.bfloat16)


def _preattn(x, ln1, w_down, w_q_up, w_k_up, w_v_up):
    out = [jax.ShapeDtypeStruct((T, D), jnp.bfloat16)] * 3
    return pl.pallas_call(
        _preattn_body,
        grid=(T // RB,),
        in_specs=[
            pl.BlockSpec((RB, D), lambda i: (i, 0)),
            pl.BlockSpec((1, D), lambda i: (0, 0)),
            pl.BlockSpec((QC + 2 * KC, D), lambda i: (0, 0)),
            pl.BlockSpec((D, QC), lambda i: (0, 0)),
            pl.BlockSpec((D, KC), lambda i: (0, 0)),
            pl.BlockSpec((D, KC), lambda i: (0, 0)),
        ],
        out_specs=[pl.BlockSpec((RB, D), lambda i: (i, 0))] * 3,
        out_shape=out,
    )(x, ln1, w_down, w_q_up, w_k_up, w_v_up)


TQ = 512


def _mm_nt16(a, b):
    return _mm_nt(a, b).astype(jnp.bfloat16)


def _attn_chunk_body(kw, q_ref, k_ref, v_ref, o_ref):
    li = lax.broadcasted_iota(jnp.int32, (TQ, TQ), 0)
    lj = lax.broadcasted_iota(jnp.int32, (TQ, TQ), 1)
    dmask = lj <= li
    outs = []
    for hh in range(2):
        sl = slice(hh * DH, (hh + 1) * DH)
        q = q_ref[:, sl]
        sd = _mm_nt16(q, k_ref[kw - TQ:kw, sl]) * SCALE
        sd = jnp.where(dmask, sd, jnp.bfloat16(-1e9))
        md = jnp.max(sd, axis=-1, keepdims=True)
        if kw > TQ:
            s0 = _mm_nt16(q, k_ref[:kw - TQ, sl]) * SCALE
            m = jnp.maximum(md, jnp.max(s0, axis=-1, keepdims=True))
            p0 = jnp.exp(s0 - m)
            pd = jnp.exp(sd - m)
            l = (jnp.sum(p0, axis=-1, keepdims=True, dtype=jnp.float32)
                 + jnp.sum(pd, axis=-1, keepdims=True, dtype=jnp.float32))
            o = (_mm_nn(p0, v_ref[:kw - TQ, sl])
                 + _mm_nn(pd, v_ref[kw - TQ:kw, sl]))
        else:
            pd = jnp.exp(sd - md)
            l = jnp.sum(pd, axis=-1, keepdims=True, dtype=jnp.float32)
            o = _mm_nn(pd, v_ref[:kw, sl])
        outs.append(o * (1.0 / l))
    o_ref[...] = jnp.concatenate(outs, axis=1).astype(jnp.bfloat16)


def _attention(q, k, v):
    parts = []
    for i in range(T // TQ):
        kw = (i + 1) * TQ
        part = pl.pallas_call(
            functools.partial(_attn_chunk_body, kw),
            grid=(H // 2,),
            in_specs=[
                pl.BlockSpec((TQ, 2 * DH), lambda h, _i=i: (_i, h)),
                pl.BlockSpec((kw, 2 * DH), lambda h: (0, h)),
                pl.BlockSpec((kw, 2 * DH), lambda h: (0, h)),
            ],
            out_specs=pl.BlockSpec((TQ, 2 * DH), lambda h: (0, h)),
            out_shape=jax.ShapeDtypeStruct((TQ, D), jnp.bfloat16),
        )(q, k, v)
        parts.append(part)
    return jnp.concatenate(parts, axis=0)


def _outproj_body(attn_ref, x_ref, wo_ref, ln2_ref, rw_ref,
                  hid_ref, h2_ref, lg_ref):
    ao = _mm_nt(attn_ref[...], wo_ref[...])
    hid = x_ref[...] + ao * RES_MULT
    hid_ref[...] = hid
    h2 = hid * lax.rsqrt(jnp.mean(hid * hid, axis=-1, keepdims=True) + EPS) * ln2_ref[...]
    h2_ref[...] = h2.astype(jnp.bfloat16)
    lg_ref[...] = lax.dot_general(h2, rw_ref[...], (((1,), (1,)), ((), ())),
                                  preferred_element_type=jnp.float32)


def _outproj(attn, x, w_o, ln2, router_w):
    out = [jax.ShapeDtypeStruct((T, D), jnp.float32),
           jax.ShapeDtypeStruct((T, D), jnp.bfloat16),
           jax.ShapeDtypeStruct((T, E), jnp.float32)]
    return pl.pallas_call(
        _outproj_body,
        grid=(T // RB,),
        in_specs=[
            pl.BlockSpec((RB, D), lambda i: (i, 0)),
            pl.BlockSpec((RB, D), lambda i: (i, 0)),
            pl.BlockSpec((D, D), lambda i: (0, 0)),
            pl.BlockSpec((1, D), lambda i: (0, 0)),
            pl.BlockSpec((E, D), lambda i: (0, 0)),
        ],
        out_specs=[
            pl.BlockSpec((RB, D), lambda i: (i, 0)),
            pl.BlockSpec((RB, D), lambda i: (i, 0)),
            pl.BlockSpec((RB, E), lambda i: (i, 0)),
        ],
        out_shape=out,
    )(attn, x, w_o, ln2, router_w)


def _route_body(lg_ref, slots_ref, cw_ref, oh_ref, cs_ref):
    lg = lg_ref[...]
    ei = lax.broadcasted_iota(jnp.int32, (T, E), 1)
    m1 = jnp.max(lg, axis=1, keepdims=True)
    a1 = jnp.min(jnp.where(lg == m1, ei, E), axis=1, keepdims=True)
    o1 = ei == a1
    lg2 = jnp.where(o1, -jnp.inf, lg)
    m2 = jnp.max(lg2, axis=1, keepdims=True)
    a2 = jnp.min(jnp.where(lg2 == m2, ei, E), axis=1, keepdims=True)
    o2 = ei == a2
    e2 = jnp.exp(m2 - m1)
    w1v = 1.0 / (1.0 + e2)
    w2v = e2 * w1v

    oh_ref[...] = o1.astype(jnp.float32) + o2.astype(jnp.float32)
    li = lax.broadcasted_iota(jnp.int32, (CB, CB), 0)
    lj = lax.broadcasted_iota(jnp.int32, (CB, CB), 1)
    ltri = (lj < li).astype(jnp.float32)

    def step(i, carry):
        ohc = oh_ref[pl.ds(i * CB, CB), :]
        local = lax.dot_general(ltri, ohc, (((1,), (0,)), ((), ())),
                                preferred_element_type=jnp.float32)
        cs_ref[pl.ds(i * CB, CB), :] = local + carry
        return carry + jnp.sum(ohc, axis=0, keepdims=True)

    lax.fori_loop(0, T // CB, step, jnp.zeros((1, E), jnp.float32))
    csum = cs_ref[...]

    pos1 = jnp.sum(jnp.where(o1, csum, 0.0), axis=1).astype(jnp.int32)
    pos2 = jnp.sum(jnp.where(o2, csum, 0.0), axis=1).astype(jnp.int32)
    a1s = a1[:, 0]
    a2s = a2[:, 0]
    val1 = pos1 < CAP
    val2 = pos2 < CAP
    gat1 = a1s * CAP + jnp.minimum(pos1, CAP - 1)
    gat2 = a2s * CAP + jnp.minimum(pos2, CAP - 1)
    sca1 = jnp.where(val1, gat1, DUMMY)
    sca2 = jnp.where(val2, gat2, DUMMY)
    slots_ref[...] = jnp.stack([sca1, sca2, gat1, gat2], axis=0)
    cw_ref[...] = jnp.stack([jnp.where(val1, w1v[:, 0], 0.0),
                             jnp.where(val2, w2v[:, 0], 0.0)], axis=0)


def _route(logits):
    return pl.pallas_call(
        _route_body,
        out_shape=[jax.ShapeDtypeStruct((4, T), jnp.int32),
                   jax.ShapeDtypeStruct((2, T), jnp.float32)],
        scratch_shapes=[pltpu.VMEM((T, E), jnp.float32),
                        pltpu.VMEM((T, E), jnp.float32)],
    )(logits)


D2 = D // 2


def _as_f32(x):
    n, d = x.shape
    return lax.bitcast_convert_type(x.reshape(n, d // 2, 2), jnp.float32)


def _as_bf16(x):
    n, d2 = x.shape
    return lax.bitcast_convert_type(x, jnp.bfloat16).reshape(n, d2 * 2)


def _dispatch_sc(h2, slots):
    mesh = plsc.VectorSubcoreMesh(core_axis_name="c", subcore_axis_name="s")

    @functools.partial(
        pl.kernel,
        out_type=jax.ShapeDtypeStruct((NBUF, D2), jnp.float32),
        mesh=mesh,
        scratch_types=[
            pltpu.VMEM((CHUNK,), jnp.int32),
            pltpu.VMEM((CHUNK, D2), jnp.float32),
            pltpu.SemaphoreType.DMA,
        ],
    )
    def run(h2_hbm, slots_hbm, buf_hbm, idx_v, rows_v, sem):
        wid = lax.axis_index("s") * 2 + lax.axis_index("c")
        base = wid * CHUNK
        pltpu.sync_copy(h2_hbm.at[pl.ds(base, CHUNK)], rows_v)
        pltpu.sync_copy(slots_hbm.at[0, pl.ds(base, CHUNK)], idx_v)
        pltpu.async_copy(rows_v, buf_hbm.at[idx_v], sem).wait()
        pltpu.sync_copy(slots_hbm.at[1, pl.ds(base, CHUNK)], idx_v)
        pltpu.async_copy(rows_v, buf_hbm.at[idx_v], sem).wait()

    return _as_bf16(run(_as_f32(h2), slots))


def _ffn_body(buf_ref, w1_ref, w3_ref, w2_ref, out_ref):
    xb = buf_ref[...]
    h1 = _mm_nt(xb, w1_ref[0])
    h3 = _mm_nt(xb, w3_ref[0])
    act = h1 * jax.nn.sigmoid(h1) * h3
    out_ref[...] = _mm_nt(act, w2_ref[0]).astype(jnp.bfloat16)


def _ffn(buf, w1, w3, w2):
    return pl.pallas_call(
        _ffn_body,
        grid=(E,),
        in_specs=[
            pl.BlockSpec((CAP, D), lambda e: (e, 0)),
            pl.BlockSpec((1, F, D), lambda e: (e, 0, 0)),
            pl.BlockSpec((1, F, D), lambda e: (e, 0, 0)),
            pl.BlockSpec((1, D, F), lambda e: (e, 0, 0)),
        ],
        out_specs=pl.BlockSpec((CAP, D), lambda e: (e, 0)),
        out_shape=jax.ShapeDtypeStruct((E * CAP, D), jnp.bfloat16),
    )(buf, w1, w3, w2)


def _combine_sc(eout, slots):
    mesh = plsc.VectorSubcoreMesh(core_axis_name="c", subcore_axis_name="s")

    @functools.partial(
        pl.kernel,
        out_type=[jax.ShapeDtypeStruct((T, D2), jnp.float32),
                  jax.ShapeDtypeStruct((T, D2), jnp.float32)],
        mesh=mesh,
        scratch_types=[
            pltpu.VMEM((CHUNK,), jnp.int32),
            pltpu.VMEM((CHUNK, D2), jnp.float32),
            pltpu.SemaphoreType.DMA,
        ],
    )
    def run(eout_hbm, slots_hbm, g1_hbm, g2_hbm, idx_v, rows_v, sem):
        wid = lax.axis_index("s") * 2 + lax.axis_index("c")
        base = wid * CHUNK
        pltpu.sync_copy(slots_hbm.at[2, pl.ds(base, CHUNK)], idx_v)
        pltpu.async_copy(eout_hbm.at[idx_v], rows_v, sem).wait()
        pltpu.sync_copy(rows_v, g1_hbm.at[pl.ds(base, CHUNK)])
        pltpu.sync_copy(slots_hbm.at[3, pl.ds(base, CHUNK)], idx_v)
        pltpu.async_copy(eout_hbm.at[idx_v], rows_v, sem).wait()
        pltpu.sync_copy(rows_v, g2_hbm.at[pl.ds(base, CHUNK)])

    g1, g2 = run(_as_f32(eout), slots)
    return _as_bf16(g1), _as_bf16(g2)


def _final_body(hid_ref, g1_ref, g2_ref, cw1_ref, cw2_ref, o_ref):
    cw1 = cw1_ref[...]
    cw2 = cw2_ref[...]
    y = (jnp.where(cw1 > 0, cw1 * g1_ref[...].astype(jnp.float32), 0.0)
         + jnp.where(cw2 > 0, cw2 * g2_ref[...].astype(jnp.float32), 0.0))
    o_ref[...] = hid_ref[...] + y * RES_MULT


def _final(hidden, g1, g2, cw1, cw2):
    return pl.pallas_call(
        _final_body,
        grid=(T // RB,),
        in_specs=[
            pl.BlockSpec((RB, D), lambda i: (i, 0)),
            pl.BlockSpec((RB, D), lambda i: (i, 0)),
            pl.BlockSpec((RB, D), lambda i: (i, 0)),
            pl.BlockSpec((RB, 1), lambda i: (i, 0)),
            pl.BlockSpec((RB, 1), lambda i: (i, 0)),
        ],
        out_specs=pl.BlockSpec((RB, D), lambda i: (i, 0)),
        out_shape=jax.ShapeDtypeStruct((T, D), jnp.float32),
    )(hidden, g1, g2, cw1, cw2)


def kernel(positions, hidden_states, w_down, w_q_up, w_k_up, w_v_up, w_o,
           ln1, ln2, router_w, w1, w3, w2):
    x = hidden_states.reshape(T, D)
    q, k, v = _preattn(x, ln1.reshape(1, D), w_down, w_q_up, w_k_up, w_v_up)
    attn = _attention(q, k, v)
    hidden, h2, logits = _outproj(attn, x, w_o, ln2.reshape(1, D), router_w)
    slots, cw = _route(logits)
    buf = _dispatch_sc(h2, slots)
    eout = _ffn(buf, w1, w3, w2)
    g1, g2 = _combine_sc(eout, slots)
    out = _final(hidden, g1, g2, cw[0][:, None], cw[1][:, None])
    return out.reshape(B, S, D)


# speedup vs baseline: 2.3039x; 2.3039x over previous
import functools

import jax
import jax.numpy as jnp
from jax import lax
from jax.experimental import pallas as pl
from jax.experimental.pallas import tpu as pltpu
from jax.experimental.pallas import tpu_sc as plsc

B, S, D = 1, 2048, 1024
H, DH = 16, 64
QC, KC = 512, 256
E, K, F = 64, 2, 512
CAP = 128
EPS = 1e-6
RES_MULT = 0.22
SCALE = 0.125
T = B * S

RB = 256
QB = 256
KB = 256
NBUF = E * CAP + CAP
DUMMY = E * CAP
CB = 256

NW = 32
CHUNK = T // NW


def _mm_nt(a, b):
    return lax.dot_general(a.astype(jnp.bfloat16), b.astype(jnp.bfloat16),
                           (((1,), (1,)), ((), ())),
                           preferred_element_type=jnp.float32)


def _mm_nn(a, b):
    return lax.dot_general(a.astype(jnp.bfloat16), b.astype(jnp.bfloat16),
                           (((1,), (0,)), ((), ())),
                           preferred_element_type=jnp.float32)


def _preattn_body(x_ref, ln1_ref, wd_ref, wq_ref, wk_ref, wv_ref,
                  q_ref, k_ref, v_ref):
    x = x_ref[...]
    h = x * lax.rsqrt(jnp.mean(x * x, axis=-1, keepdims=True) + EPS) * ln1_ref[...]
    c = _mm_nt(h, wd_ref[...])
    q_ref[...] = _mm_nt(c[:, :QC], wq_ref[...]).astype(jnp.bfloat16)
    k_ref[...] = _mm_nt(c[:, QC:QC + KC], wk_ref[...]).astype(jnp.bfloat16)
    v_ref[...] = _mm_nt(c[:, QC + KC:], wv_ref[...]).astype(jnp.bfloat16)


def _preattn(x, ln1, w_down, w_q_up, w_k_up, w_v_up):
    out = [jax.ShapeDtypeStruct((T, D), jnp.bfloat16)] * 3
    return pl.pallas_call(
        _preattn_body,
        grid=(T // RB,),
        in_specs=[
            pl.BlockSpec((RB, D), lambda i: (i, 0)),
            pl.BlockSpec((1, D), lambda i: (0, 0)),
            pl.BlockSpec((QC + 2 * KC, D), lambda i: (0, 0)),
            pl.BlockSpec((D, QC), lambda i: (0, 0)),
            pl.BlockSpec((D, KC), lambda i: (0, 0)),
            pl.BlockSpec((D, KC), lambda i: (0, 0)),
        ],
        out_specs=[pl.BlockSpec((RB, D), lambda i: (i, 0))] * 3,
        out_shape=out,
    )(x, ln1, w_down, w_q_up, w_k_up, w_v_up)


TQ = 512


def _mm_nt16(a, b):
    return _mm_nt(a, b).astype(jnp.bfloat16)


def _attn_chunk_body(kw, q_ref, k_ref, v_ref, o_ref):
    li = lax.broadcasted_iota(jnp.int32, (TQ, TQ), 0)
    lj = lax.broadcasted_iota(jnp.int32, (TQ, TQ), 1)
    dmask = lj <= li
    outs = []
    for hh in range(2):
        sl = slice(hh * DH, (hh + 1) * DH)
        q = q_ref[:, sl]
        sd = _mm_nt16(q, k_ref[kw - TQ:kw, sl]) * SCALE
        sd = jnp.where(dmask, sd, jnp.bfloat16(-1e9))
        md = jnp.max(sd, axis=-1, keepdims=True)
        if kw > TQ:
            s0 = _mm_nt16(q, k_ref[:kw - TQ, sl]) * SCALE
            m = jnp.maximum(md, jnp.max(s0, axis=-1, keepdims=True))
            p0 = jnp.exp(s0 - m)
            pd = jnp.exp(sd - m)
            l = (jnp.sum(p0, axis=-1, keepdims=True, dtype=jnp.float32)
                 + jnp.sum(pd, axis=-1, keepdims=True, dtype=jnp.float32))
            o = (_mm_nn(p0, v_ref[:kw - TQ, sl])
                 + _mm_nn(pd, v_ref[kw - TQ:kw, sl]))
        else:
            pd = jnp.exp(sd - md)
            l = jnp.sum(pd, axis=-1, keepdims=True, dtype=jnp.float32)
            o = _mm_nn(pd, v_ref[:kw, sl])
        outs.append(o * (1.0 / l))
    o_ref[...] = jnp.concatenate(outs, axis=1).astype(jnp.bfloat16)


def _attention(q, k, v):
    parts = []
    for i in range(T // TQ):
        kw = (i + 1) * TQ
        part = pl.pallas_call(
            functools.partial(_attn_chunk_body, kw),
            grid=(H // 2,),
            in_specs=[
                pl.BlockSpec((TQ, 2 * DH), lambda h, _i=i: (_i, h)),
                pl.BlockSpec((kw, 2 * DH), lambda h: (0, h)),
                pl.BlockSpec((kw, 2 * DH), lambda h: (0, h)),
            ],
            out_specs=pl.BlockSpec((TQ, 2 * DH), lambda h: (0, h)),
            out_shape=jax.ShapeDtypeStruct((TQ, D), jnp.bfloat16),
        )(q, k, v)
        parts.append(part)
    return jnp.concatenate(parts, axis=0)


def _outproj_body(attn_ref, x_ref, wo_ref, ln2_ref, rw_ref,
                  hid_ref, h2_ref, lg_ref):
    ao = _mm_nt(attn_ref[...], wo_ref[...])
    hid = x_ref[...] + ao * RES_MULT
    hid_ref[...] = hid
    h2 = hid * lax.rsqrt(jnp.mean(hid * hid, axis=-1, keepdims=True) + EPS) * ln2_ref[...]
    h2_ref[...] = h2
    lg_ref[...] = lax.dot_general(h2, rw_ref[...], (((1,), (1,)), ((), ())),
                                  preferred_element_type=jnp.float32)


def _outproj(attn, x, w_o, ln2, router_w):
    out = [jax.ShapeDtypeStruct((T, D), jnp.float32),
           jax.ShapeDtypeStruct((T, D), jnp.float32),
           jax.ShapeDtypeStruct((T, E), jnp.float32)]
    return pl.pallas_call(
        _outproj_body,
        grid=(T // RB,),
        in_specs=[
            pl.BlockSpec((RB, D), lambda i: (i, 0)),
            pl.BlockSpec((RB, D), lambda i: (i, 0)),
            pl.BlockSpec((D, D), lambda i: (0, 0)),
            pl.BlockSpec((1, D), lambda i: (0, 0)),
            pl.BlockSpec((E, D), lambda i: (0, 0)),
        ],
        out_specs=[
            pl.BlockSpec((RB, D), lambda i: (i, 0)),
            pl.BlockSpec((RB, D), lambda i: (i, 0)),
            pl.BlockSpec((RB, E), lambda i: (i, 0)),
        ],
        out_shape=out,
    )(attn, x, w_o, ln2, router_w)


def _route_body(lg_ref, slots_ref, cw_ref, oh_ref, cs_ref):
    lg = lg_ref[...]
    ei = lax.broadcasted_iota(jnp.int32, (T, E), 1)
    m1 = jnp.max(lg, axis=1, keepdims=True)
    a1 = jnp.min(jnp.where(lg == m1, ei, E), axis=1, keepdims=True)
    o1 = ei == a1
    lg2 = jnp.where(o1, -jnp.inf, lg)
    m2 = jnp.max(lg2, axis=1, keepdims=True)
    a2 = jnp.min(jnp.where(lg2 == m2, ei, E), axis=1, keepdims=True)
    o2 = ei == a2
    e2 = jnp.exp(m2 - m1)
    w1v = 1.0 / (1.0 + e2)
    w2v = e2 * w1v

    oh_ref[...] = o1.astype(jnp.float32) + o2.astype(jnp.float32)
    li = lax.broadcasted_iota(jnp.int32, (CB, CB), 0)
    lj = lax.broadcasted_iota(jnp.int32, (CB, CB), 1)
    ltri = (lj < li).astype(jnp.float32)

    def step(i, carry):
        ohc = oh_ref[pl.ds(i * CB, CB), :]
        local = lax.dot_general(ltri, ohc, (((1,), (0,)), ((), ())),
                                preferred_element_type=jnp.float32)
        cs_ref[pl.ds(i * CB, CB), :] = local + carry
        return carry + jnp.sum(ohc, axis=0, keepdims=True)

    lax.fori_loop(0, T // CB, step, jnp.zeros((1, E), jnp.float32))
    csum = cs_ref[...]

    pos1 = jnp.sum(jnp.where(o1, csum, 0.0), axis=1).astype(jnp.int32)
    pos2 = jnp.sum(jnp.where(o2, csum, 0.0), axis=1).astype(jnp.int32)
    a1s = a1[:, 0]
    a2s = a2[:, 0]
    val1 = pos1 < CAP
    val2 = pos2 < CAP
    gat1 = a1s * CAP + jnp.minimum(pos1, CAP - 1)
    gat2 = a2s * CAP + jnp.minimum(pos2, CAP - 1)
    sca1 = jnp.where(val1, gat1, DUMMY)
    sca2 = jnp.where(val2, gat2, DUMMY)
    slots_ref[...] = jnp.stack([sca1, sca2, gat1, gat2], axis=0)
    cw_ref[...] = jnp.stack([jnp.where(val1, w1v[:, 0], 0.0),
                             jnp.where(val2, w2v[:, 0], 0.0)], axis=0)


def _route(logits):
    return pl.pallas_call(
        _route_body,
        out_shape=[jax.ShapeDtypeStruct((4, T), jnp.int32),
                   jax.ShapeDtypeStruct((2, T), jnp.float32)],
        scratch_shapes=[pltpu.VMEM((T, E), jnp.float32),
                        pltpu.VMEM((T, E), jnp.float32)],
    )(logits)


def _dispatch_sc(h2, slots):
    mesh = plsc.VectorSubcoreMesh(core_axis_name="c", subcore_axis_name="s")

    @functools.partial(
        pl.kernel,
        out_type=jax.ShapeDtypeStruct((NBUF, D), jnp.float32),
        mesh=mesh,
        scratch_types=[
            pltpu.VMEM((CHUNK,), jnp.int32),
            pltpu.VMEM((CHUNK, D), jnp.float32),
            pltpu.SemaphoreType.DMA,
        ],
    )
    def run(h2_hbm, slots_hbm, buf_hbm, idx_v, rows_v, sem):
        wid = lax.axis_index("s") * 2 + lax.axis_index("c")
        base = wid * CHUNK
        pltpu.sync_copy(h2_hbm.at[pl.ds(base, CHUNK)], rows_v)
        pltpu.sync_copy(slots_hbm.at[0, pl.ds(base, CHUNK)], idx_v)
        pltpu.async_copy(rows_v, buf_hbm.at[idx_v], sem).wait()
        pltpu.sync_copy(slots_hbm.at[1, pl.ds(base, CHUNK)], idx_v)
        pltpu.async_copy(rows_v, buf_hbm.at[idx_v], sem).wait()

    return run(h2, slots)


def _ffn_body(buf_ref, w1_ref, w3_ref, w2_ref, out_ref):
    xb = buf_ref[...]
    h1 = _mm_nt(xb, w1_ref[0])
    h3 = _mm_nt(xb, w3_ref[0])
    act = h1 * jax.nn.sigmoid(h1) * h3
    out_ref[...] = _mm_nt(act, w2_ref[0])


def _ffn(buf, w1, w3, w2):
    return pl.pallas_call(
        _ffn_body,
        grid=(E,),
        in_specs=[
            pl.BlockSpec((CAP, D), lambda e: (e, 0)),
            pl.BlockSpec((1, F, D), lambda e: (e, 0, 0)),
            pl.BlockSpec((1, F, D), lambda e: (e, 0, 0)),
            pl.BlockSpec((1, D, F), lambda e: (e, 0, 0)),
        ],
        out_specs=pl.BlockSpec((CAP, D), lambda e: (e, 0)),
        out_shape=jax.ShapeDtypeStruct((E * CAP, D), jnp.float32),
    )(buf, w1, w3, w2)


def _combine_sc(eout, slots):
    mesh = plsc.VectorSubcoreMesh(core_axis_name="c", subcore_axis_name="s")

    @functools.partial(
        pl.kernel,
        out_type=[jax.ShapeDtypeStruct((T, D), jnp.float32),
                  jax.ShapeDtypeStruct((T, D), jnp.float32)],
        mesh=mesh,
        scratch_types=[
            pltpu.VMEM((CHUNK,), jnp.int32),
            pltpu.VMEM((CHUNK, D), jnp.float32),
            pltpu.SemaphoreType.DMA,
        ],
    )
    def run(eout_hbm, slots_hbm, g1_hbm, g2_hbm, idx_v, rows_v, sem):
        wid = lax.axis_index("s") * 2 + lax.axis_index("c")
        base = wid * CHUNK
        pltpu.sync_copy(slots_hbm.at[2, pl.ds(base, CHUNK)], idx_v)
        pltpu.async_copy(eout_hbm.at[idx_v], rows_v, sem).wait()
        pltpu.sync_copy(rows_v, g1_hbm.at[pl.ds(base, CHUNK)])
        pltpu.sync_copy(slots_hbm.at[3, pl.ds(base, CHUNK)], idx_v)
        pltpu.async_copy(eout_hbm.at[idx_v], rows_v, sem).wait()
        pltpu.sync_copy(rows_v, g2_hbm.at[pl.ds(base, CHUNK)])

    return run(eout, slots)


def _final_body(hid_ref, g1_ref, g2_ref, cw1_ref, cw2_ref, o_ref):
    cw1 = cw1_ref[...]
    cw2 = cw2_ref[...]
    y = (jnp.where(cw1 > 0, cw1 * g1_ref[...], 0.0)
         + jnp.where(cw2 > 0, cw2 * g2_ref[...], 0.0))
    o_ref[...] = hid_ref[...] + y * RES_MULT


def _final(hidden, g1, g2, cw1, cw2):
    return pl.pallas_call(
        _final_body,
        grid=(T // RB,),
        in_specs=[
            pl.BlockSpec((RB, D), lambda i: (i, 0)),
            pl.BlockSpec((RB, D), lambda i: (i, 0)),
            pl.BlockSpec((RB, D), lambda i: (i, 0)),
            pl.BlockSpec((RB, 1), lambda i: (i, 0)),
            pl.BlockSpec((RB, 1), lambda i: (i, 0)),
        ],
        out_specs=pl.BlockSpec((RB, D), lambda i: (i, 0)),
        out_shape=jax.ShapeDtypeStruct((T, D), jnp.float32),
    )(hidden, g1, g2, cw1, cw2)


def kernel(positions, hidden_states, w_down, w_q_up, w_k_up, w_v_up, w_o,
           ln1, ln2, router_w, w1, w3, w2):
    x = hidden_states.reshape(T, D)
    q, k, v = _preattn(x, ln1.reshape(1, D), w_down, w_q_up, w_k_up, w_v_up)
    attn = _attention(q, k, v)
    hidden, h2, logits = _outproj(attn, x, w_o, ln2.reshape(1, D), router_w)
    slots, cw = _route(logits)
    buf = _dispatch_sc(h2, slots)
    eout = _ffn(buf, w1, w3, w2)
    g1, g2 = _combine_sc(eout, slots)
    out = _final(hidden, g1, g2, cw[0][:, None], cw[1][:, None])
    return out.reshape(B, S, D)
